# baseline (speedup 1.0000x reference)
import functools

import jax
import jax.numpy as jnp
from jax import lax
from jax.experimental import pallas as pl
from jax.experimental.pallas import tpu as pltpu
from jax.experimental.pallas import tpu_sc as plsc

NUM_CORES = 2
NUM_SUBCORES = 16
NW = NUM_CORES * NUM_SUBCORES

CHUNK = 128


def _gather_kernel(n_total, d, idx_hbm, table_hbm, out_hbm, idx_v, rows_v, sems):
    per_w = n_total // NW
    n_chunks = per_w // CHUNK
    wid = lax.axis_index("s") * NUM_CORES + lax.axis_index("c")
    base = wid * per_w

    pltpu.sync_copy(idx_hbm.at[pl.ds(base, per_w)], idx_v)

    def start_gather(j, buf):
        pltpu.async_copy(
            table_hbm.at[idx_v.at[pl.ds(j * CHUNK, CHUNK)]],
            rows_v.at[buf],
            sems.at[buf],
        )

    def wait_and_store(j, buf):
        pltpu.make_async_copy(
            table_hbm.at[idx_v.at[pl.ds(j * CHUNK, CHUNK)]],
            rows_v.at[buf],
            sems.at[buf],
        ).wait()
        pltpu.sync_copy(rows_v.at[buf], out_hbm.at[pl.ds(base + j * CHUNK, CHUNK)])

    start_gather(0, 0)
    start_gather(1, 1)

    def body(j2, _):
        j = j2 * 2
        wait_and_store(j, 0)

        @pl.when(j + 2 < n_chunks)
        def _():
            start_gather(j + 2, 0)

        wait_and_store(j + 1, 1)

        @pl.when(j + 3 < n_chunks)
        def _():
            start_gather(j + 3, 1)

        return 0

    lax.fori_loop(0, n_chunks // 2, body, 0)


def kernel(item_ids, table):
    batch, hist = item_ids.shape
    n_total = batch * hist
    d = table.shape[1]
    idx = item_ids.reshape(n_total)

    mesh = plsc.VectorSubcoreMesh(
        core_axis_name="c",
        subcore_axis_name="s",
        num_cores=NUM_CORES,
        num_subcores=NUM_SUBCORES,
    )
    per_w = n_total // NW

    grid_kernel = pl.kernel(
        functools.partial(_gather_kernel, n_total, d),
        out_type=jax.ShapeDtypeStruct((n_total, d), table.dtype),
        mesh=mesh,
        scratch_types=[
            pltpu.VMEM((per_w,), jnp.int32),
            pltpu.VMEM((2, CHUNK, d), table.dtype),
            pltpu.SemaphoreType.DMA((2,)),
        ],
        compiler_params=pltpu.CompilerParams(use_tc_tiling_on_sc=False),
    )
    out = grid_kernel(idx, table)
    return out.reshape(batch, hist, d)


# speedup vs baseline: 1.0207x; 1.0207x over previous
import functools

import jax
import jax.numpy as jnp
from jax import lax
from jax.experimental import pallas as pl
from jax.experimental.pallas import tpu as pltpu
from jax.experimental.pallas import tpu_sc as plsc

NUM_CORES = 2
NUM_SUBCORES = 16
NW = NUM_CORES * NUM_SUBCORES

CHUNK = 128


NBUF = 8


def _gather_kernel(n_total, d, idx_hbm, table_hbm, out_hbm, idx_v, rows_v,
                   gsems, ssems):
    per_w = n_total // NW
    n_chunks = per_w // CHUNK
    n_groups = n_chunks // NBUF
    wid = lax.axis_index("s") * NUM_CORES + lax.axis_index("c")
    base = wid * per_w

    pltpu.sync_copy(idx_hbm.at[pl.ds(base, per_w)], idx_v)

    def start_gather(j, buf):
        pltpu.async_copy(
            table_hbm.at[idx_v.at[pl.ds(j * CHUNK, CHUNK)]],
            rows_v.at[buf],
            gsems.at[buf],
        )

    def wait_gather(j, buf):
        pltpu.make_async_copy(
            table_hbm.at[idx_v.at[pl.ds(j * CHUNK, CHUNK)]],
            rows_v.at[buf],
            gsems.at[buf],
        ).wait()

    def start_store(j, buf):
        pltpu.async_copy(
            rows_v.at[buf],
            out_hbm.at[pl.ds(base + j * CHUNK, CHUNK)],
            ssems.at[buf],
        )

    def wait_store(j, buf):
        pltpu.make_async_copy(
            rows_v.at[buf],
            out_hbm.at[pl.ds(base + j * CHUNK, CHUNK)],
            ssems.at[buf],
        ).wait()

    for b in range(NBUF):
        start_gather(b, b)

    def body(g, _):
        j0 = g * NBUF
        for b in range(NBUF):
            wait_gather(j0 + b, b)
            start_store(j0 + b, b)

        @pl.when(g + 1 < n_groups)
        def _():
            for b in range(NBUF):
                wait_store(j0 + b, b)
                start_gather(j0 + NBUF + b, b)

        return 0

    lax.fori_loop(0, n_groups, body, 0)

    for b in range(NBUF):
        wait_store((n_groups - 1) * NBUF + b, b)


def kernel(item_ids, table):
    batch, hist = item_ids.shape
    n_total = batch * hist
    d = table.shape[1]
    idx = item_ids.reshape(n_total)

    mesh = plsc.VectorSubcoreMesh(
        core_axis_name="c",
        subcore_axis_name="s",
        num_cores=NUM_CORES,
        num_subcores=NUM_SUBCORES,
    )
    per_w = n_total // NW

    grid_kernel = pl.kernel(
        functools.partial(_gather_kernel, n_total, d),
        out_type=jax.ShapeDtypeStruct((n_total, d), table.dtype),
        mesh=mesh,
        scratch_types=[
            pltpu.VMEM((per_w,), jnp.int32),
            pltpu.VMEM((NBUF, CHUNK, d), table.dtype),
            pltpu.SemaphoreType.DMA((NBUF,)),
            pltpu.SemaphoreType.DMA((NBUF,)),
        ],
        compiler_params=pltpu.CompilerParams(use_tc_tiling_on_sc=False),
    )
    out = grid_kernel(idx, table)
    return out.reshape(batch, hist, d)
